# Initial kernel scaffold; baseline (speedup 1.0000x reference)
import jax
import jax.numpy as jnp
from jax.experimental import pallas as pl


def kernel(x, table, bias):
    raise NotImplementedError("write your pallas kernel here")


# speedup vs baseline: 1.4214x; 1.4214x over previous
import functools

import jax
import jax.numpy as jnp
from jax import lax
from jax.experimental import pallas as pl
from jax.experimental.pallas import tpu as pltpu
from jax.experimental.pallas import tpu_sc as plsc

F = 26
V_PER_F = 40000
B = 16384
NC = 2
NS = 16
NW = NC * NS
BPW = B // NW
LANES = 16
IPW = BPW * F
CHUNK = 128
NCHUNK = IPW // CHUNK

_mesh = plsc.VectorSubcoreMesh(core_axis_name="c", subcore_axis_name="s")


@functools.partial(
    pl.kernel,
    out_type=jax.ShapeDtypeStruct((B,), jnp.float32),
    mesh=_mesh,
    scratch_types=[
        pltpu.VMEM((IPW,), jnp.int32),
        pltpu.VMEM((IPW,), jnp.int32),
        pltpu.VMEM((IPW,), jnp.float32),
        pltpu.VMEM((BPW,), jnp.float32),
        pltpu.SemaphoreType.DMA,
        pltpu.SemaphoreType.DMA,
    ],
)
def _cat_linear_sc(xt_hbm, table_hbm, out_hbm, xv, idxv, vals, outv,
                   sem_x, sem_g):
    cid = lax.axis_index("c")
    sid = lax.axis_index("s")
    wid = sid * NC + cid
    base = wid * BPW

    def fire_x(f, _):
        pltpu.make_async_copy(
            xt_hbm.at[pl.ds(f * B + base, BPW)],
            xv.at[pl.ds(f * BPW, BPW)],
            sem_x,
        ).start()
        return 0
    lax.fori_loop(0, F, fire_x, 0)
    pltpu.make_async_copy(xt_hbm.at[pl.ds(0, IPW)], xv, sem_x).wait()

    def build_f(f, _):
        off = f * V_PER_F
        fb = f * BPW
        def build_j(j, _):
            p = fb + j * LANES
            idxv[pl.ds(p, LANES)] = xv[pl.ds(p, LANES)] + off
            return 0
        return lax.fori_loop(0, BPW // LANES, build_j, 0)
    lax.fori_loop(0, F, build_f, 0)

    def fire_g(c, _):
        sl = pl.ds(c * CHUNK, CHUNK)
        pltpu.make_async_copy(
            table_hbm.at[idxv.at[sl]], vals.at[sl], sem_g,
        ).start()
        return 0
    lax.fori_loop(0, NCHUNK, fire_g, 0)
    pltpu.make_async_copy(table_hbm.at[pl.ds(0, IPW)], vals, sem_g).wait()

    def red_j(j, _):
        jb = j * LANES
        def red_f(f, acc):
            return acc + vals[pl.ds(f * BPW + jb, LANES)]
        outv[pl.ds(jb, LANES)] = lax.fori_loop(
            0, F, red_f, jnp.zeros((LANES,), jnp.float32))
        return 0
    lax.fori_loop(0, BPW // LANES, red_j, 0)

    pltpu.sync_copy(outv, out_hbm.at[pl.ds(base, BPW)])


@jax.jit
def kernel(x, table, bias):
    xt = x.T.reshape(F * B)
    out = _cat_linear_sc(xt, table.reshape(-1))
    return out.reshape(B, 1) + bias[None, :]
